# baseline (speedup 1.0000x reference)
import functools

import jax
import jax.numpy as jnp
import numpy as np
from jax.experimental import pallas as pl

B, N, K, H = 32, 256, 32, 8
D = 128
CROWD = 5
CH = 256
NCH = N // CH
E = CH * K


def _edge_kernel(h_ref, pos_ref, vel_ref, acc_ref, crowd_ref, hist_ref,
                 maskE_ref, ideE_ref,
                 SelT_ref, Xp_ref, XpT_ref,
                 W1h_ref, W1s6T_ref, b1_ref, W2T_ref, b2_ref,
                 W3h_ref, W3aT_ref, W3cT_ref, b3_ref, lng_ref, lnb_ref,
                 wt2c_ref, S48T_ref, w8_ref, out_ref):
    f32 = jnp.float32
    h = h_ref[0]
    posT = pos_ref[0].T
    velT = vel_ref[0].T
    accT = acc_ref[0].T
    crowdT = crowd_ref[0].T
    histT = hist_ref[0].T
    tdims = (((0,), (1,)), ((), ()))

    hW1T = (jax.lax.dot_general(W1h_ref[...], h, tdims,
                                preferred_element_type=f32) + b1_ref[...])
    histAT = histT * wt2c_ref[...]
    qT = jnp.dot(S48T_ref[...], histAT * histT,
                 preferred_element_type=f32)
    pvT = jnp.concatenate([posT, velT], axis=0)
    T2T = jnp.concatenate([histT, qT], axis=0)

    pmT = jnp.concatenate([velT, accT], axis=0)
    ped_norm = jnp.sqrt(jnp.sum(pmT * pmT, 0, keepdims=True))
    cmT = crowdT[0:4]
    crowd_norm = jnp.sqrt(jnp.sum(cmT * cmT, 0, keepdims=True))
    dotpc = jnp.sum(pmT * cmT, 0, keepdims=True)
    csimT = (dotpc / (ped_norm * crowd_norm + 1e-6) + 1.0) * 0.5

    mu = jnp.mean(crowdT, 0, keepdims=True)
    var = jnp.mean((crowdT - mu) ** 2, 0, keepdims=True)
    crowd1T = ((crowdT - mu) * jax.lax.rsqrt(var + 1e-5) * lng_ref[...]
               + lnb_ref[...])
    node_baseT = (jax.lax.dot_general(W3h_ref[...], h, tdims,
                                      preferred_element_type=f32)
                  + jnp.dot(W3cT_ref[...], crowd1T, preferred_element_type=f32)
                  + b3_ref[...])

    SelT = SelT_ref[0]
    Xp = Xp_ref[...]
    XpT = XpT_ref[...]

    TBLq = jnp.concatenate([qT, histAT, pvT, csimT], axis=0)
    QcT = jnp.dot(TBLq, SelT, preferred_element_type=f32)
    QeT = jnp.dot(QcT.astype(jnp.bfloat16), Xp,
                  preferred_element_type=f32)
    qiT = QeT[0:8]
    histAiT = QeT[8:56]
    posiT = QeT[56:58]
    veliT = QeT[58:60]
    csimiT = QeT[60:61]

    m = maskE_ref[0, 0]
    ide = ideE_ref[0, 0]
    idx = (ide.astype(f32) * m).astype(jnp.int16)
    jiota = jax.lax.broadcasted_iota(jnp.int16, (N, E), 0)
    OH1T = (jiota == idx).astype(jnp.bfloat16)
    OH2T = (jiota == ide).astype(jnp.bfloat16)
    TBL1 = jnp.concatenate([hW1T, pvT], axis=0).astype(jnp.bfloat16)
    G1 = jnp.dot(TBL1, OH1T, preferred_element_type=f32)
    g2 = jnp.dot(T2T.astype(jnp.bfloat16), OH2T,
                 preferred_element_type=f32)
    g1T = G1[0:D]
    pvjT = G1[D:D + 4]
    histjT = g2[0:48]
    qjT = g2[48:56]

    relT = pvjT[0:2] - posiT
    distT = jnp.sqrt(jnp.sum(relT * relT, 0, keepdims=True)) + 1e-6
    dvT = veliT - pvjT[2:4]
    rspeedT = jnp.sqrt(jnp.sum(dvT * dvT, 0, keepdims=True))
    crossT = jnp.dot(S48T_ref[...], histAiT * histjT,
                     preferred_element_type=f32)
    d2 = jnp.maximum(qiT + qjT - 2.0 * crossT, 0.0)
    simtT = jnp.exp(-jnp.sqrt(d2))
    hsimT = jnp.dot(w8_ref[...], simtT, preferred_element_type=f32) * 0.1
    scalT = jnp.concatenate([relT, distT, csimiT, hsimT, rspeedT], axis=0)

    zsT = jnp.dot(W1s6T_ref[...], scalT.astype(jnp.bfloat16),
                  preferred_element_type=f32)
    e1T = (jnp.maximum(g1T + zsT, 0.0).astype(jnp.bfloat16)
           * m.astype(jnp.bfloat16))
    sT = jnp.dot(e1T, XpT, preferred_element_type=f32)
    msum = jnp.dot(m.astype(jnp.bfloat16), XpT,
                   preferred_element_type=f32)
    aggT = ((jnp.dot(W2T_ref[...], sT, preferred_element_type=f32)
             + b2_ref[...] * msum) / (msum + 1e-6))

    nbT = jnp.dot(node_baseT, SelT, preferred_element_type=f32)
    oT = jnp.maximum(nbT + jnp.dot(W3aT_ref[...], aggT,
                                   preferred_element_type=f32), 0.0)
    out_ref[0] = oT.T


@jax.jit
def kernel(h, pos, vel, acc, crowd, mask, idex, hist_feature,
           W1, b1, W2, b2, W3, b3, ln_g, ln_b):
    f32 = jnp.float32
    hist = hist_feature.reshape(B, N, H * 6)

    W1h = W1[:D]
    W1s6T = W1[D:D + 6].T.astype(jnp.bfloat16)
    W2T = W2.T
    W3h = W3[:D]
    W3aT = W3[D:2 * D].T
    W3cT = W3[2 * D:2 * D + CROWD].T
    wt = np.array([0.1, 0.1, 1.0, 1.0, 0.5, 0.5], np.float32)
    wt2c = jnp.asarray(np.tile(wt * wt, H).reshape(H * 6, 1))
    S48T = jnp.asarray(np.kron(np.eye(H, dtype=np.float32),
                               np.ones((1, 6), np.float32)))
    wts = 0.8 ** np.arange(H - 1, -1, -1, dtype=np.float32)
    w8 = jnp.asarray((wts / (wts.sum() + 1e-6)).reshape(1, H))

    maskE = mask.reshape(B, NCH, 1, E)
    ideE = idex.astype(jnp.int16).reshape(B, NCH, 1, E)

    narange = np.arange(N, dtype=np.int32)
    erange = np.arange(E, dtype=np.int32) // K
    SelT_np = (narange[None, :, None] ==
               (np.arange(NCH, dtype=np.int32)[:, None, None] * CH
                + np.arange(CH, dtype=np.int32)[None, None, :])
               ).astype(np.float32)
    Xp_np = (np.arange(CH, dtype=np.int32)[:, None] == erange[None, :])
    SelT_in = jnp.asarray(SelT_np)
    Xp_in = jnp.asarray(Xp_np.astype(np.float32)).astype(jnp.bfloat16)
    XpT_in = jnp.asarray(Xp_np.T.astype(np.float32)).astype(jnp.bfloat16)

    grid = (B, NCH)
    bcast = lambda shape: pl.BlockSpec(shape, lambda b, c: (0,) * len(shape))
    perb = lambda shape: pl.BlockSpec((1,) + shape, lambda b, c: (b, 0, 0))
    edge = pl.BlockSpec((1, 1, 1, E), lambda b, c: (b, c, 0, 0))
    out = pl.pallas_call(
        _edge_kernel,
        grid=grid,
        in_specs=[
            perb((N, D)),
            perb((N, 2)), perb((N, 2)), perb((N, 2)),
            perb((N, CROWD)),
            perb((N, H * 6)),
            edge, edge,
            pl.BlockSpec((1, N, CH), lambda b, c: (c, 0, 0)),
            bcast((CH, E)), bcast((E, CH)),
            bcast((D, D)), bcast((D, 6)), bcast((D, 1)),
            bcast((D, D)), bcast((D, 1)),
            bcast((D, D)), bcast((D, D)), bcast((D, CROWD)), bcast((D, 1)),
            bcast((CROWD, 1)), bcast((CROWD, 1)),
            bcast((H * 6, 1)), bcast((H, H * 6)), bcast((1, H)),
        ],
        out_specs=pl.BlockSpec((1, CH, D), lambda b, c: (b, c, 0)),
        out_shape=jax.ShapeDtypeStruct((B, N, D), f32),
    )(h, pos, vel, acc, crowd, hist,
      maskE, ideE,
      SelT_in, Xp_in, XpT_in,
      W1h, W1s6T, b1.reshape(D, 1), W2T, b2.reshape(D, 1),
      W3h, W3aT, W3cT, b3.reshape(D, 1),
      ln_g.reshape(CROWD, 1), ln_b.reshape(CROWD, 1), wt2c, S48T, w8)
    return out


# speedup vs baseline: 1.3042x; 1.3042x over previous
import functools

import jax
import jax.numpy as jnp
import numpy as np
from jax.experimental import pallas as pl

B, N, K, H = 32, 256, 32, 8
D = 128
CROWD = 5
CH = 256
NCH = N // CH
E = CH * K


def _edge_kernel(h_ref, pos_ref, vel_ref, acc_ref, crowd_ref, hist_ref,
                 maskE_ref, ideE_ref,
                 SelT_ref, Xp_ref, XpT_ref,
                 W1h_ref, W1s6T_ref, b1_ref, W2T_ref, b2_ref,
                 W3h_ref, W3aT_ref, W3cT_ref, b3_ref, lng_ref, lnb_ref,
                 wt2c_ref, S48T_ref, w8_ref, out_ref):
    f32 = jnp.float32
    h = h_ref[0]
    posT = pos_ref[0].T
    velT = vel_ref[0].T
    accT = acc_ref[0].T
    crowdT = crowd_ref[0].T
    histT = hist_ref[0].T
    tdims = (((0,), (1,)), ((), ()))

    hW1T = (jax.lax.dot_general(W1h_ref[...], h, tdims,
                                preferred_element_type=f32) + b1_ref[...])
    histAT = histT * wt2c_ref[...]
    qT = jnp.dot(S48T_ref[...], histAT * histT,
                 preferred_element_type=f32)
    pvT = jnp.concatenate([posT, velT], axis=0)
    T2T = jnp.concatenate([histT, qT], axis=0)

    pmT = jnp.concatenate([velT, accT], axis=0)
    ped_norm = jnp.sqrt(jnp.sum(pmT * pmT, 0, keepdims=True))
    cmT = crowdT[0:4]
    crowd_norm = jnp.sqrt(jnp.sum(cmT * cmT, 0, keepdims=True))
    dotpc = jnp.sum(pmT * cmT, 0, keepdims=True)
    csimT = (dotpc / (ped_norm * crowd_norm + 1e-6) + 1.0) * 0.5

    mu = jnp.mean(crowdT, 0, keepdims=True)
    var = jnp.mean((crowdT - mu) ** 2, 0, keepdims=True)
    crowd1T = ((crowdT - mu) * jax.lax.rsqrt(var + 1e-5) * lng_ref[...]
               + lnb_ref[...])
    node_baseT = (jax.lax.dot_general(W3h_ref[...], h, tdims,
                                      preferred_element_type=f32)
                  + jnp.dot(W3cT_ref[...], crowd1T, preferred_element_type=f32)
                  + b3_ref[...])

    SelT = SelT_ref[0]
    Xp = Xp_ref[...]
    XpT = XpT_ref[...]

    TBLq = jnp.concatenate([qT, histAT, pvT, csimT], axis=0)
    QcT = jnp.dot(TBLq, SelT, preferred_element_type=f32)
    QeT = jnp.dot(QcT.astype(jnp.bfloat16), Xp,
                  preferred_element_type=f32)
    qiT = QeT[0:8]
    histAiT = QeT[8:56]
    posiT = QeT[56:58]
    veliT = QeT[58:60]
    csimiT = QeT[60:61]

    m = maskE_ref[0, 0]
    ide = ideE_ref[0, 0]
    idx = (ide.astype(f32) * m).astype(jnp.int32)
    jiota = jax.lax.broadcasted_iota(jnp.int32, (N, E), 0)
    OH1T = (jiota == idx).astype(jnp.bfloat16)
    OH2T = (jiota == ide).astype(jnp.bfloat16)
    TBL1 = jnp.concatenate([hW1T, pvT], axis=0).astype(jnp.bfloat16)
    G1 = jnp.dot(TBL1, OH1T, preferred_element_type=f32)
    g2 = jnp.dot(T2T.astype(jnp.bfloat16), OH2T,
                 preferred_element_type=f32)
    g1T = G1[0:D]
    pvjT = G1[D:D + 4]
    histjT = g2[0:48]
    qjT = g2[48:56]

    relT = pvjT[0:2] - posiT
    distT = jnp.sqrt(jnp.sum(relT * relT, 0, keepdims=True)) + 1e-6
    dvT = veliT - pvjT[2:4]
    rspeedT = jnp.sqrt(jnp.sum(dvT * dvT, 0, keepdims=True))
    crossT = jnp.dot(S48T_ref[...], histAiT * histjT,
                     preferred_element_type=f32)
    d2 = jnp.maximum(qiT + qjT - 2.0 * crossT, 0.0)
    simtT = jnp.exp(-jnp.sqrt(d2))
    hsimT = jnp.dot(w8_ref[...], simtT, preferred_element_type=f32) * 0.1
    scalT = jnp.concatenate([relT, distT, csimiT, hsimT, rspeedT], axis=0)

    zsT = jnp.dot(W1s6T_ref[...], scalT.astype(jnp.bfloat16),
                  preferred_element_type=f32)
    e1T = (jnp.maximum(g1T + zsT, 0.0).astype(jnp.bfloat16)
           * m.astype(jnp.bfloat16))
    sT = jnp.dot(e1T, XpT, preferred_element_type=f32)
    msum = jnp.dot(m.astype(jnp.bfloat16), XpT,
                   preferred_element_type=f32)
    aggT = ((jnp.dot(W2T_ref[...], sT, preferred_element_type=f32)
             + b2_ref[...] * msum) / (msum + 1e-6))

    nbT = jnp.dot(node_baseT, SelT, preferred_element_type=f32)
    oT = jnp.maximum(nbT + jnp.dot(W3aT_ref[...], aggT,
                                   preferred_element_type=f32), 0.0)
    out_ref[0] = oT.T


@jax.jit
def kernel(h, pos, vel, acc, crowd, mask, idex, hist_feature,
           W1, b1, W2, b2, W3, b3, ln_g, ln_b):
    f32 = jnp.float32
    hist = hist_feature.reshape(B, N, H * 6)

    W1h = W1[:D]
    W1s6T = W1[D:D + 6].T.astype(jnp.bfloat16)
    W2T = W2.T
    W3h = W3[:D]
    W3aT = W3[D:2 * D].T
    W3cT = W3[2 * D:2 * D + CROWD].T
    wt = np.array([0.1, 0.1, 1.0, 1.0, 0.5, 0.5], np.float32)
    wt2c = jnp.asarray(np.tile(wt * wt, H).reshape(H * 6, 1))
    S48T = jnp.asarray(np.kron(np.eye(H, dtype=np.float32),
                               np.ones((1, 6), np.float32)))
    wts = 0.8 ** np.arange(H - 1, -1, -1, dtype=np.float32)
    w8 = jnp.asarray((wts / (wts.sum() + 1e-6)).reshape(1, H))

    maskE = mask.reshape(B, NCH, 1, E)
    ideE = idex.reshape(B, NCH, 1, E)

    narange = np.arange(N, dtype=np.int32)
    erange = np.arange(E, dtype=np.int32) // K
    SelT_np = (narange[None, :, None] ==
               (np.arange(NCH, dtype=np.int32)[:, None, None] * CH
                + np.arange(CH, dtype=np.int32)[None, None, :])
               ).astype(np.float32)
    Xp_np = (np.arange(CH, dtype=np.int32)[:, None] == erange[None, :])
    SelT_in = jnp.asarray(SelT_np)
    Xp_in = jnp.asarray(Xp_np.astype(np.float32)).astype(jnp.bfloat16)
    XpT_in = jnp.asarray(Xp_np.T.astype(np.float32)).astype(jnp.bfloat16)

    grid = (B, NCH)
    bcast = lambda shape: pl.BlockSpec(shape, lambda b, c: (0,) * len(shape))
    perb = lambda shape: pl.BlockSpec((1,) + shape, lambda b, c: (b, 0, 0))
    edge = pl.BlockSpec((1, 1, 1, E), lambda b, c: (b, c, 0, 0))
    out = pl.pallas_call(
        _edge_kernel,
        grid=grid,
        in_specs=[
            perb((N, D)),
            perb((N, 2)), perb((N, 2)), perb((N, 2)),
            perb((N, CROWD)),
            perb((N, H * 6)),
            edge, edge,
            pl.BlockSpec((1, N, CH), lambda b, c: (c, 0, 0)),
            bcast((CH, E)), bcast((E, CH)),
            bcast((D, D)), bcast((D, 6)), bcast((D, 1)),
            bcast((D, D)), bcast((D, 1)),
            bcast((D, D)), bcast((D, D)), bcast((D, CROWD)), bcast((D, 1)),
            bcast((CROWD, 1)), bcast((CROWD, 1)),
            bcast((H * 6, 1)), bcast((H, H * 6)), bcast((1, H)),
        ],
        out_specs=pl.BlockSpec((1, CH, D), lambda b, c: (b, c, 0)),
        out_shape=jax.ShapeDtypeStruct((B, N, D), f32),
    )(h, pos, vel, acc, crowd, hist,
      maskE, ideE,
      SelT_in, Xp_in, XpT_in,
      W1h, W1s6T, b1.reshape(D, 1), W2T, b2.reshape(D, 1),
      W3h, W3aT, W3cT, b3.reshape(D, 1),
      ln_g.reshape(CROWD, 1), ln_b.reshape(CROWD, 1), wt2c, S48T, w8)
    return out


# speedup vs baseline: 1.9770x; 1.5159x over previous
import functools

import jax
import jax.numpy as jnp
import numpy as np
from jax.experimental import pallas as pl

B, N, K, H = 32, 256, 32, 8
D = 128
CROWD = 5
E = N * K


def _edge_kernel(h_ref, pos_ref, vel_ref, acc_ref, crowd_ref, hist_ref,
                 maskE_ref, ideE_ref,
                 W1h_ref, W1s6T_ref, b1_ref, W2T_ref, b2_ref,
                 W3h_ref, W3aT_ref, W3cT_ref, b3_ref, lng_ref, lnb_ref,
                 wt2c_ref, S48T_ref, w8_ref, out_ref):
    f32 = jnp.float32
    h = h_ref[0]
    posT = pos_ref[0].T
    velT = vel_ref[0].T
    accT = acc_ref[0].T
    crowdT = crowd_ref[0].T
    histT = hist_ref[0].T
    tdims = (((0,), (1,)), ((), ()))

    hW1T = (jax.lax.dot_general(W1h_ref[...], h, tdims,
                                preferred_element_type=f32) + b1_ref[...])
    histAT = histT * wt2c_ref[...]
    qT = jnp.dot(S48T_ref[...], histAT * histT,
                 preferred_element_type=f32)
    pvT = jnp.concatenate([posT, velT], axis=0)
    T2T = jnp.concatenate([histT, qT], axis=0)

    pmT = jnp.concatenate([velT, accT], axis=0)
    ped_norm = jnp.sqrt(jnp.sum(pmT * pmT, 0, keepdims=True))
    cmT = crowdT[0:4]
    crowd_norm = jnp.sqrt(jnp.sum(cmT * cmT, 0, keepdims=True))
    dotpc = jnp.sum(pmT * cmT, 0, keepdims=True)
    csimT = (dotpc / (ped_norm * crowd_norm + 1e-6) + 1.0) * 0.5

    mu = jnp.mean(crowdT, 0, keepdims=True)
    var = jnp.mean((crowdT - mu) ** 2, 0, keepdims=True)
    crowd1T = ((crowdT - mu) * jax.lax.rsqrt(var + 1e-5) * lng_ref[...]
               + lnb_ref[...])
    node_baseT = (jax.lax.dot_general(W3h_ref[...], h, tdims,
                                      preferred_element_type=f32)
                  + jnp.dot(W3cT_ref[...], crowd1T, preferred_element_type=f32)
                  + b3_ref[...])

    TBLq = jnp.concatenate([qT, histAT, pvT, csimT], axis=0)
    QeT = jnp.concatenate([TBLq] * K, axis=1)
    qiT = QeT[0:8]
    histAiT = QeT[8:56]
    posiT = QeT[56:58]
    veliT = QeT[58:60]
    csimiT = QeT[60:61]

    m = maskE_ref[0]
    ide = ideE_ref[0]
    idx = (ide.astype(f32) * m).astype(jnp.int32)
    jiota = jax.lax.broadcasted_iota(jnp.int32, (N, E), 0)
    OH1T = (jiota == idx).astype(jnp.bfloat16)
    OH2T = (jiota == ide).astype(jnp.bfloat16)
    TBL1 = jnp.concatenate([hW1T, pvT], axis=0).astype(jnp.bfloat16)
    G1 = jnp.dot(TBL1, OH1T, preferred_element_type=f32)
    g2 = jnp.dot(T2T.astype(jnp.bfloat16), OH2T,
                 preferred_element_type=f32)
    g1T = G1[0:D]
    pvjT = G1[D:D + 4]
    histjT = g2[0:48]
    qjT = g2[48:56]

    relT = pvjT[0:2] - posiT
    distT = jnp.sqrt(jnp.sum(relT * relT, 0, keepdims=True)) + 1e-6
    dvT = veliT - pvjT[2:4]
    rspeedT = jnp.sqrt(jnp.sum(dvT * dvT, 0, keepdims=True))
    crossT = jnp.dot(S48T_ref[...], histAiT * histjT,
                     preferred_element_type=f32)
    d2 = jnp.maximum(qiT + qjT - 2.0 * crossT, 0.0)
    simtT = jnp.exp(-jnp.sqrt(d2))
    hsimT = jnp.dot(w8_ref[...], simtT, preferred_element_type=f32) * 0.1
    scalT = jnp.concatenate([relT, distT, csimiT, hsimT, rspeedT], axis=0)

    zsT = jnp.dot(W1s6T_ref[...], scalT.astype(jnp.bfloat16),
                  preferred_element_type=f32)
    e1m = jnp.maximum(g1T + zsT, 0.0) * m
    sT = e1m
    mrow = m
    w = E
    while w > N:
        w //= 2
        sT = sT[:, :w] + sT[:, w:2 * w]
        mrow = mrow[:, :w] + mrow[:, w:2 * w]
    msum = mrow
    aggT = ((jnp.dot(W2T_ref[...], sT, preferred_element_type=f32)
             + b2_ref[...] * msum) / (msum + 1e-6))

    oT = jnp.maximum(node_baseT + jnp.dot(W3aT_ref[...], aggT,
                                          preferred_element_type=f32), 0.0)
    out_ref[0] = oT.T


@jax.jit
def kernel(h, pos, vel, acc, crowd, mask, idex, hist_feature,
           W1, b1, W2, b2, W3, b3, ln_g, ln_b):
    f32 = jnp.float32
    hist = hist_feature.reshape(B, N, H * 6)

    W1h = W1[:D]
    W1s6T = W1[D:D + 6].T.astype(jnp.bfloat16)
    W2T = W2.T
    W3h = W3[:D]
    W3aT = W3[D:2 * D].T
    W3cT = W3[2 * D:2 * D + CROWD].T
    wt = np.array([0.1, 0.1, 1.0, 1.0, 0.5, 0.5], np.float32)
    wt2c = jnp.asarray(np.tile(wt * wt, H).reshape(H * 6, 1))
    S48T = jnp.asarray(np.kron(np.eye(H, dtype=np.float32),
                               np.ones((1, 6), np.float32)))
    wts = 0.8 ** np.arange(H - 1, -1, -1, dtype=np.float32)
    w8 = jnp.asarray((wts / (wts.sum() + 1e-6)).reshape(1, H))

    maskE = jnp.swapaxes(mask, 1, 2).reshape(B, 1, E)
    ideE = jnp.swapaxes(idex, 1, 2).reshape(B, 1, E)

    grid = (B,)
    bcast = lambda shape: pl.BlockSpec(shape, lambda b: (0,) * len(shape))
    perb = lambda shape: pl.BlockSpec((1,) + shape, lambda b: (b, 0, 0))
    edge = pl.BlockSpec((1, 1, E), lambda b: (b, 0, 0))
    out = pl.pallas_call(
        _edge_kernel,
        grid=grid,
        in_specs=[
            perb((N, D)),
            perb((N, 2)), perb((N, 2)), perb((N, 2)),
            perb((N, CROWD)),
            perb((N, H * 6)),
            edge, edge,
            bcast((D, D)), bcast((D, 6)), bcast((D, 1)),
            bcast((D, D)), bcast((D, 1)),
            bcast((D, D)), bcast((D, D)), bcast((D, CROWD)), bcast((D, 1)),
            bcast((CROWD, 1)), bcast((CROWD, 1)),
            bcast((H * 6, 1)), bcast((H, H * 6)), bcast((1, H)),
        ],
        out_specs=pl.BlockSpec((1, N, D), lambda b: (b, 0, 0)),
        out_shape=jax.ShapeDtypeStruct((B, N, D), f32),
    )(h, pos, vel, acc, crowd, hist,
      maskE, ideE,
      W1h, W1s6T, b1.reshape(D, 1), W2T, b2.reshape(D, 1),
      W3h, W3aT, W3cT, b3.reshape(D, 1),
      ln_g.reshape(CROWD, 1), ln_b.reshape(CROWD, 1), wt2c, S48T, w8)
    return out


# speedup vs baseline: 2.0320x; 1.0279x over previous
import functools

import jax
import jax.numpy as jnp
import numpy as np
from jax.experimental import pallas as pl

B, N, K, H = 32, 256, 32, 8
D = 128
CROWD = 5
E = N * K


def _edge_kernel(h_ref, pos_ref, vel_ref, acc_ref, crowd_ref, hist_ref,
                 maskE_ref, ideE_ref,
                 W1h_ref, W1s6T_ref, b1_ref, W2T_ref, b2_ref,
                 W3h_ref, W3aT_ref, W3cT_ref, b3_ref, lng_ref, lnb_ref,
                 wt2c_ref, w8_ref, out_ref):
    f32 = jnp.float32
    h = h_ref[0]
    posT = pos_ref[0].T
    velT = vel_ref[0].T
    accT = acc_ref[0].T
    crowdT = crowd_ref[0].T
    histT = hist_ref[0].T
    tdims = (((0,), (1,)), ((), ()))

    hW1T = (jax.lax.dot_general(W1h_ref[...], h, tdims,
                                preferred_element_type=f32) + b1_ref[...])
    histAT = histT * wt2c_ref[...]
    A2 = histAT * histT
    qT = (A2[0:8] + A2[8:16] + A2[16:24] + A2[24:32]
          + A2[32:40] + A2[40:48])
    pvT = jnp.concatenate([posT, velT], axis=0)
    T2T = jnp.concatenate([histT, qT], axis=0)

    pmT = jnp.concatenate([velT, accT], axis=0)
    ped_norm = jnp.sqrt(jnp.sum(pmT * pmT, 0, keepdims=True))
    cmT = crowdT[0:4]
    crowd_norm = jnp.sqrt(jnp.sum(cmT * cmT, 0, keepdims=True))
    dotpc = jnp.sum(pmT * cmT, 0, keepdims=True)
    csimT = (dotpc / (ped_norm * crowd_norm + 1e-6) + 1.0) * 0.5

    mu = jnp.mean(crowdT, 0, keepdims=True)
    var = jnp.mean((crowdT - mu) ** 2, 0, keepdims=True)
    crowd1T = ((crowdT - mu) * jax.lax.rsqrt(var + 1e-5) * lng_ref[...]
               + lnb_ref[...])
    node_baseT = (jax.lax.dot_general(W3h_ref[...], h, tdims,
                                      preferred_element_type=f32)
                  + jnp.dot(W3cT_ref[...], crowd1T, preferred_element_type=f32)
                  + b3_ref[...])

    TBLq = jnp.concatenate([qT, histAT, pvT, csimT], axis=0)
    QeT = jnp.concatenate([TBLq] * K, axis=1)
    qiT = QeT[0:8]
    histAiT = QeT[8:56]
    posiT = QeT[56:58]
    veliT = QeT[58:60]
    csimiT = QeT[60:61]

    m = maskE_ref[0]
    ide = ideE_ref[0]
    idx = (ide.astype(f32) * m).astype(jnp.int32)
    jiota = jax.lax.broadcasted_iota(jnp.int32, (N, E), 0)
    OH1T = (jiota == idx).astype(jnp.bfloat16)
    OH2T = (jiota == ide).astype(jnp.bfloat16)
    TBL1 = jnp.concatenate([hW1T, pvT], axis=0).astype(jnp.bfloat16)
    G1 = jnp.dot(TBL1, OH1T, preferred_element_type=f32)
    g2 = jnp.dot(T2T.astype(jnp.bfloat16), OH2T,
                 preferred_element_type=f32)
    g1T = G1[0:D]
    pvjT = G1[D:D + 4]
    histjT = g2[0:48]
    qjT = g2[48:56]

    relT = pvjT[0:2] - posiT
    distT = jnp.sqrt(jnp.sum(relT * relT, 0, keepdims=True)) + 1e-6
    dvT = veliT - pvjT[2:4]
    rspeedT = jnp.sqrt(jnp.sum(dvT * dvT, 0, keepdims=True))
    P = histAiT * histjT
    crossT = (P[0:8] + P[8:16] + P[16:24] + P[24:32]
              + P[32:40] + P[40:48])
    d2 = jnp.maximum(qiT + qjT - 2.0 * crossT, 0.0)
    simtw = jnp.exp(-jnp.sqrt(d2)) * w8_ref[...]
    s4 = simtw[0:4] + simtw[4:8]
    s2 = s4[0:2] + s4[2:4]
    hsimT = (s2[0:1] + s2[1:2]) * 0.1
    scalT = jnp.concatenate([relT, distT, csimiT, hsimT, rspeedT], axis=0)

    zsT = jnp.dot(W1s6T_ref[...], scalT.astype(jnp.bfloat16),
                  preferred_element_type=f32)
    e1m = jnp.maximum(g1T + zsT, 0.0) * m
    sT = e1m
    mrow = m
    w = E
    while w > N:
        w //= 2
        sT = sT[:, :w] + sT[:, w:2 * w]
        mrow = mrow[:, :w] + mrow[:, w:2 * w]
    msum = mrow
    aggT = ((jnp.dot(W2T_ref[...], sT, preferred_element_type=f32)
             + b2_ref[...] * msum) / (msum + 1e-6))

    oT = jnp.maximum(node_baseT + jnp.dot(W3aT_ref[...], aggT,
                                          preferred_element_type=f32), 0.0)
    out_ref[0] = oT.T


@jax.jit
def kernel(h, pos, vel, acc, crowd, mask, idex, hist_feature,
           W1, b1, W2, b2, W3, b3, ln_g, ln_b):
    f32 = jnp.float32
    hist = hist_feature.transpose(0, 1, 3, 2).reshape(B, N, H * 6)

    W1h = W1[:D]
    W1s6T = W1[D:D + 6].T.astype(jnp.bfloat16)
    W2T = W2.T
    W3h = W3[:D]
    W3aT = W3[D:2 * D].T
    W3cT = W3[2 * D:2 * D + CROWD].T
    wt = np.array([0.1, 0.1, 1.0, 1.0, 0.5, 0.5], np.float32)
    wt2c = jnp.asarray(np.repeat(wt * wt, H).reshape(H * 6, 1))
    wts = 0.8 ** np.arange(H - 1, -1, -1, dtype=np.float32)
    w8 = jnp.asarray((wts / (wts.sum() + 1e-6)).reshape(H, 1))

    maskE = jnp.swapaxes(mask, 1, 2).reshape(B, 1, E)
    ideE = jnp.swapaxes(idex, 1, 2).reshape(B, 1, E)

    grid = (B,)
    bcast = lambda shape: pl.BlockSpec(shape, lambda b: (0,) * len(shape))
    perb = lambda shape: pl.BlockSpec((1,) + shape, lambda b: (b, 0, 0))
    edge = pl.BlockSpec((1, 1, E), lambda b: (b, 0, 0))
    out = pl.pallas_call(
        _edge_kernel,
        grid=grid,
        in_specs=[
            perb((N, D)),
            perb((N, 2)), perb((N, 2)), perb((N, 2)),
            perb((N, CROWD)),
            perb((N, H * 6)),
            edge, edge,
            bcast((D, D)), bcast((D, 6)), bcast((D, 1)),
            bcast((D, D)), bcast((D, 1)),
            bcast((D, D)), bcast((D, D)), bcast((D, CROWD)), bcast((D, 1)),
            bcast((CROWD, 1)), bcast((CROWD, 1)),
            bcast((H * 6, 1)), bcast((H, 1)),
        ],
        out_specs=pl.BlockSpec((1, N, D), lambda b: (b, 0, 0)),
        out_shape=jax.ShapeDtypeStruct((B, N, D), f32),
    )(h, pos, vel, acc, crowd, hist,
      maskE, ideE,
      W1h, W1s6T, b1.reshape(D, 1), W2T, b2.reshape(D, 1),
      W3h, W3aT, W3cT, b3.reshape(D, 1),
      ln_g.reshape(CROWD, 1), ln_b.reshape(CROWD, 1), wt2c, w8)
    return out
